# baseline (speedup 1.0000x reference)
import functools

import jax
import jax.numpy as jnp
from jax.experimental import pallas as pl
from jax.experimental.pallas import tpu as pltpu


def _project_body(table_ref, w_ref, b_ref, out_ref):
    out_ref[...] = (
        jnp.dot(table_ref[...], w_ref[...], preferred_element_type=jnp.float32)
        + b_ref[...]
    )


def _project(table, W, b):
    V, _ = table.shape
    MD = W.shape[1]
    return pl.pallas_call(
        _project_body,
        out_shape=jax.ShapeDtypeStruct((V, MD), jnp.float32),
    )(table, W, b.reshape(1, MD))


def _make_row_gather(V, MD, B, L, TB):
    def body(idx_ref, p_ref, out_ref):
        for bb in range(TB):
            for l in range(L):
                i = idx_ref[0, bb, l]
                out_ref[bb, l, :] = p_ref[i, :]

    return pl.pallas_call(
        body,
        grid=(B // TB,),
        in_specs=[
            pl.BlockSpec((1, TB, L), lambda b: (b, 0, 0),
                         memory_space=pltpu.SMEM),
            pl.BlockSpec((V, MD), lambda b: (0, 0)),
        ],
        out_specs=pl.BlockSpec((TB, L, MD), lambda b: (b, 0, 0)),
        out_shape=jax.ShapeDtypeStruct((B, L, MD), jnp.float32),
    )


def kernel(x, table, W, b):
    B, L = x.shape
    V, D = table.shape
    MD = W.shape[1]
    P = _project(table, W, b)
    TB = 2
    x3 = x.astype(jnp.int32).reshape(B // TB, TB, L)
    return _make_row_gather(V, MD, B, L, TB)(x3, P)


# speedup vs baseline: 1.5340x; 1.5340x over previous
import functools

import jax
import jax.numpy as jnp
from jax.experimental import pallas as pl
from jax.experimental.pallas import tpu as pltpu


def _project_body(table_ref, w_ref, b_ref, out_ref):
    out_ref[...] = (
        jnp.dot(table_ref[...], w_ref[...], preferred_element_type=jnp.float32)
        + b_ref[...]
    )


def _project(table, W, b):
    V, _ = table.shape
    MD = W.shape[1]
    return pl.pallas_call(
        _project_body,
        out_shape=jax.ShapeDtypeStruct((V, MD), jnp.float32),
    )(table, W, b.reshape(1, MD))


def _make_row_gather(V, MD, B, L, TB):
    def body(idx_ref, p_ref, out_ref):
        for bb in range(TB):
            for l in range(L):
                i = idx_ref[0, bb, l]
                out_ref[bb, l, :] = p_ref[i, :]

    return pl.pallas_call(
        body,
        grid=(B // TB,),
        in_specs=[
            pl.BlockSpec((1, TB, L), lambda b: (b, 0, 0),
                         memory_space=pltpu.SMEM),
            pl.BlockSpec((V, MD), lambda b: (0, 0)),
        ],
        out_specs=pl.BlockSpec((TB, L, MD), lambda b: (b, 0, 0)),
        out_shape=jax.ShapeDtypeStruct((B, L, MD), jnp.float32),
    )


def kernel(x, table, W, b):
    B, L = x.shape
    V, D = table.shape
    MD = W.shape[1]
    P = _project(table, W, b)
    TB = 8
    x3 = x.astype(jnp.int32).reshape(B // TB, TB, L)
    return _make_row_gather(V, MD, B, L, TB)(x3, P)


# speedup vs baseline: 1.6867x; 1.0996x over previous
import functools

import jax
import jax.numpy as jnp
from jax.experimental import pallas as pl
from jax.experimental.pallas import tpu as pltpu


def _project_body(table_ref, w_ref, b_ref, out_ref):
    out_ref[...] = (
        jnp.dot(table_ref[...], w_ref[...], preferred_element_type=jnp.float32)
        + b_ref[...]
    )


def _project(table, W, b):
    V, _ = table.shape
    MD = W.shape[1]
    return pl.pallas_call(
        _project_body,
        out_shape=jax.ShapeDtypeStruct((V, MD), jnp.float32),
    )(table, W, b.reshape(1, MD))


def _make_row_gather(V, MD, B, L, TB):
    def body(idx_ref, p_ref, out_ref):
        for bb in range(TB):
            for l in range(L):
                i = idx_ref[0, bb, l]
                out_ref[bb, l, :] = p_ref[i, :]

    return pl.pallas_call(
        body,
        grid=(B // TB,),
        in_specs=[
            pl.BlockSpec((1, TB, L), lambda b: (b, 0, 0),
                         memory_space=pltpu.SMEM),
            pl.BlockSpec((V, MD), lambda b: (0, 0)),
        ],
        out_specs=pl.BlockSpec((TB, L, MD), lambda b: (b, 0, 0)),
        out_shape=jax.ShapeDtypeStruct((B, L, MD), jnp.float32),
        compiler_params=pltpu.CompilerParams(
            dimension_semantics=("parallel",)
        ),
    )


def kernel(x, table, W, b):
    B, L = x.shape
    V, D = table.shape
    MD = W.shape[1]
    P = _project(table, W, b)
    TB = 32
    x3 = x.astype(jnp.int32).reshape(B // TB, TB, L)
    return _make_row_gather(V, MD, B, L, TB)(x3, P)
